# baseline (speedup 1.0000x reference)
import jax
import jax.numpy as jnp
from jax import lax
from jax.experimental import pallas as pl
from jax.experimental.pallas import tpu as pltpu
from jax.experimental.pallas import tpu_sc as plsc

_N = 1000000
_D = 32
_B = 16384

_NC = 2
_NS = 16

_CW = 640
_TAIL = _N - 64
_NQ = 4
_QSPAN = 13 * _CW
_QCAP = 240
_QBUF = 256
_HBUF = _NQ * _QBUF
_CCAP = 48
_CB = _CCAP // 16
_CVALS = _CCAP * _D

_SCAN_VREGS = _B // 16
_QB_VREGS = _QCAP // 16 + 1
_OUTS_WORDS = (_B + 1) * _D


def _body(idx_hbm, tabT_hbm, p0_hbm, p1_hbm,
          abuf, buf0, buf1, buf2, tailbuf, a_hit, b_hit, ca, cb,
          cv0, cv1, cv2, ix0, ix1, ix2, zbuf, outS,
          sem0, sem1, sem2, semA, semS):
    sc = lax.axis_index("c")
    tl = lax.axis_index("s")
    wid = tl * _NC + sc

    cb_base = 244 * wid + jnp.minimum(wid, 5)
    ncols = jnp.where(wid < 5, 245, 244)
    a_lo = cb_base * 128
    a_hi = jnp.minimum((cb_base + ncols) * 128, _N)

    pltpu.make_async_copy(idx_hbm, abuf, semA).start()

    def chunk_off(i):
        return pl.multiple_of(jnp.minimum(a_lo + _CW * i, 999296), 128)

    def chunk_copy(i, buf, sem):
        return pltpu.make_async_copy(
            tabT_hbm.at[:, pl.ds(chunk_off(i), _CW)], buf, sem)

    bufs = (buf0, buf1, buf2)
    sems = (sem0, sem1, sem2)
    cvs = (cv0, cv1, cv2)
    ixs = (ix0, ix1, ix2)
    chunk_copy(0, buf0, sem0).start()
    chunk_copy(1, buf1, sem1).start()
    chunk_copy(2, buf2, sem2).start()

    zero16 = jnp.zeros((16,), jnp.float32)
    for k in range(128):
        zbuf[pl.ds(16 * k, 16)] = zero16
    for k in range(16):
        pltpu.sync_copy(zbuf, outS.at[pl.ds(32768 * tl + 2048 * k, 2048)])

    big16 = jnp.full((16,), jnp.int32(1 << 30), jnp.int32)
    dumpb16 = jnp.full((16,), jnp.int32(_B), jnp.int32)
    for k in range(_HBUF // 16):
        a_hit[pl.ds(16 * k, 16)] = big16
        b_hit[pl.ds(16 * k, 16)] = dumpb16

    plsc.subcore_barrier()

    pltpu.make_async_copy(idx_hbm, abuf, semA).wait()
    iota16 = lax.iota(jnp.int32, 16)

    def scan_step(k, cnts):
        v = abuf[pl.ds(k * 16, 16)]
        m = (v >= a_lo) & (v < a_hi)
        qv = (v - a_lo) // _QSPAN
        bvec = k * 16 + iota16
        new = []
        for q in range(_NQ):
            mq = m & (qv == q)
            cq = cnts[q]
            plsc.store_compressed(a_hit.at[pl.ds(_QBUF * q + cq, 16)], v,
                                  mask=mq)
            plsc.store_compressed(b_hit.at[pl.ds(_QBUF * q + cq, 16)], bvec,
                                  mask=mq)
            nm = plsc.all_reduce_population_count(mq)[0]
            new.append(jnp.minimum(cq + nm, _QCAP))
        return tuple(new)

    lax.fori_loop(0, _SCAN_VREGS, scan_step,
                  (jnp.int32(0),) * _NQ)

    zero16i = jnp.zeros((16,), jnp.int32)

    def process(buf, q, off, width, cv, ix):
        def rebin_step(k, cnt2):
            base = pl.ds(_QBUF * q + 16 * k, 16)
            av = a_hit[base]
            m2 = (av >= off) & (av < off + width)
            plsc.store_compressed(ca.at[pl.ds(cnt2, 16)], av - off, mask=m2)
            plsc.store_compressed(cb.at[pl.ds(cnt2, 16)], b_hit[base],
                                  mask=m2)
            nm = plsc.all_reduce_population_count(m2)[0]
            return jnp.minimum(cnt2 + nm, _CCAP)

        cnt2 = lax.fori_loop(0, _QB_VREGS, rebin_step, jnp.int32(0))
        for p in range(_CB):
            ca[pl.ds(cnt2 + 16 * p, 16)] = zero16i
            cb[pl.ds(cnt2 + 16 * p, 16)] = dumpb16

        for t in range(_CB):
            jv = ca[pl.ds(16 * t, 16)]
            bv = cb[pl.ds(16 * t, 16)]
            bv32 = bv * _D
            lpos = (16 * t + iota16) * _D
            for c in range(_D):
                cvec = jnp.full((16,), jnp.int32(c), jnp.int32)
                vals = plsc.load_gather(buf, [cvec, jv])
                plsc.store_scatter(cv, [lpos + c], vals)
                plsc.store_scatter(ix, [lpos + c], bv32 + c)

    def scatter(cv, ix):
        pltpu.make_async_copy(cv, outS.at[ix], semS).start()

    def scatter_wait(cv, ix):
        pltpu.make_async_copy(cv, outS.at[ix], semS).wait()

    dump_ix16 = jnp.full((16,), jnp.int32(_B * _D), jnp.int32)
    for k in range(_CVALS // 16):
        ix0[pl.ds(16 * k, 16)] = dump_ix16
        ix1[pl.ds(16 * k, 16)] = dump_ix16
        ix2[pl.ds(16 * k, 16)] = dump_ix16
    scatter(cv0, ix0)
    scatter(cv1, ix1)
    scatter(cv2, ix2)

    def triple_step(g, _):
        for b in range(3):
            i = 3 * g + b
            buf, sem, cv, ix = bufs[b], sems[b], cvs[b], ixs[b]
            pltpu.make_async_copy(
                tabT_hbm.at[:, pl.ds(chunk_off(i), _CW)], buf, sem).wait()
            scatter_wait(cv, ix)
            q = jnp.minimum(i * _CW // _QSPAN, _NQ - 1)
            process(buf, q, chunk_off(i), _CW, cv, ix)
            scatter(cv, ix)
            pltpu.make_async_copy(
                tabT_hbm.at[:, pl.ds(chunk_off(i + 3), _CW)], buf, sem
            ).start()
        return 0

    lax.fori_loop(0, 18, triple_step, 0)
    for bb, ss in ((buf0, sem0), (buf1, sem1), (buf2, sem2)):
        pltpu.make_async_copy(
            tabT_hbm.at[:, pl.ds(chunk_off(0), _CW)], bb, ss).wait()

    scatter_wait(cv0, ix0)
    scatter_wait(cv1, ix1)
    scatter_wait(cv2, ix2)
    pltpu.sync_copy(tabT_hbm.at[:, pl.ds(_TAIL, 64)], tailbuf)
    process(tailbuf, _NQ - 1, jnp.int32(_TAIL), 64, cv0, ix0)
    scatter(cv0, ix0)
    scatter_wait(cv0, ix0)
    plsc.subcore_barrier()

    @pl.when(sc == 0)
    def _():
        pltpu.sync_copy(outS.at[pl.ds(32768 * tl, 32768)],
                        p0_hbm.at[pl.ds(32768 * tl, 32768)])

    @pl.when(sc == 1)
    def _():
        pltpu.sync_copy(outS.at[pl.ds(32768 * tl, 32768)],
                        p1_hbm.at[pl.ds(32768 * tl, 32768)])


def kernel(action, table):
    tabT = table.T
    mesh = plsc.VectorSubcoreMesh(core_axis_name="c", subcore_axis_name="s")
    out_sds = jax.ShapeDtypeStruct((_B * _D,), jnp.float32)
    k = pl.kernel(
        _body,
        out_type=(out_sds, out_sds),
        mesh=mesh,
        scratch_types=[
            pltpu.VMEM((_B,), jnp.int32),
            pltpu.VMEM((_D, _CW), jnp.float32),
            pltpu.VMEM((_D, _CW), jnp.float32),
            pltpu.VMEM((_D, _CW), jnp.float32),
            pltpu.VMEM((_D, 64), jnp.float32),
            pltpu.VMEM((_HBUF,), jnp.int32),
            pltpu.VMEM((_HBUF,), jnp.int32),
            pltpu.VMEM((_CCAP + 48,), jnp.int32),
            pltpu.VMEM((_CCAP + 48,), jnp.int32),
            pltpu.VMEM((_CVALS,), jnp.float32),
            pltpu.VMEM((_CVALS,), jnp.float32),
            pltpu.VMEM((_CVALS,), jnp.float32),
            pltpu.VMEM((_CVALS,), jnp.int32),
            pltpu.VMEM((_CVALS,), jnp.int32),
            pltpu.VMEM((_CVALS,), jnp.int32),
            pltpu.VMEM((2048,), jnp.float32),
            pltpu.VMEM_SHARED((_OUTS_WORDS,), jnp.float32),
            pltpu.SemaphoreType.DMA,
            pltpu.SemaphoreType.DMA,
            pltpu.SemaphoreType.DMA,
            pltpu.SemaphoreType.DMA,
            pltpu.SemaphoreType.DMA,
        ],
        compiler_params=pltpu.CompilerParams(needs_layout_passes=False),
    )
    p0, p1 = k(action, tabT)
    return (p0 + p1).reshape(_B, _D)


# speedup vs baseline: 1.5711x; 1.5711x over previous
import jax
import jax.numpy as jnp
from jax import lax
from jax.experimental import pallas as pl
from jax.experimental.pallas import tpu as pltpu
from jax.experimental.pallas import tpu_sc as plsc

_N = 1000000
_D = 32
_B = 16384

_NC = 2
_NS = 16

_CW = 1024
_TAIL = _N - 64
_NQ = 4
_QSPAN = 8 * _CW
_QCAP = 240
_QBUF = 256
_HBUF = _NQ * _QBUF
_CCAP = 48
_CB = _CCAP // 16
_CVALS = _CCAP * _D

_SCAN_VREGS = _B // 16
_QB_VREGS = _QCAP // 16 + 1
_OUTS_WORDS = (_B + 1) * _D


def _body(idx_hbm, tabT_hbm, p0_hbm, p1_hbm,
          abuf, buf0, buf1, tailbuf, a_hit, b_hit, ca, cb,
          cv0, cv1, ix0, ix1, zbuf, outS,
          sem0, sem1, semA, semS):
    sc = lax.axis_index("c")
    tl = lax.axis_index("s")
    wid = tl * _NC + sc

    cb_base = 244 * wid + jnp.minimum(wid, 5)
    ncols = jnp.where(wid < 5, 245, 244)
    a_lo = cb_base * 128
    a_hi = jnp.minimum((cb_base + ncols) * 128, _N)

    pltpu.make_async_copy(idx_hbm, abuf, semA).start()

    def chunk_off(i):
        return pl.multiple_of(jnp.minimum(a_lo + _CW * i, 998912), 128)

    def chunk_copy(i, buf, sem):
        return pltpu.make_async_copy(
            tabT_hbm.at[:, pl.ds(chunk_off(i), _CW)], buf, sem)

    bufs = (buf0, buf1)
    sems = (sem0, sem1)
    chunk_copy(0, buf0, sem0).start()
    chunk_copy(1, buf1, sem1).start()

    zero16 = jnp.zeros((16,), jnp.float32)
    for k in range(128):
        zbuf[pl.ds(16 * k, 16)] = zero16
    for k in range(16):
        pltpu.sync_copy(zbuf, outS.at[pl.ds(32768 * tl + 2048 * k, 2048)])

    big16 = jnp.full((16,), jnp.int32(1 << 30), jnp.int32)
    dumpb16 = jnp.full((16,), jnp.int32(_B), jnp.int32)
    for k in range(_HBUF // 16):
        a_hit[pl.ds(16 * k, 16)] = big16
        b_hit[pl.ds(16 * k, 16)] = dumpb16

    plsc.subcore_barrier()

    pltpu.make_async_copy(idx_hbm, abuf, semA).wait()
    iota16 = lax.iota(jnp.int32, 16)

    def scan_step(k, cnts):
        v = abuf[pl.ds(k * 16, 16)]
        m = (v >= a_lo) & (v < a_hi)
        qv = lax.shift_right_logical(v - a_lo, 13)
        bvec = k * 16 + iota16
        new = []
        for q in range(_NQ):
            mq = m & (qv == q)
            cq = cnts[q]
            plsc.store_compressed(a_hit.at[pl.ds(_QBUF * q + cq, 16)], v,
                                  mask=mq)
            plsc.store_compressed(b_hit.at[pl.ds(_QBUF * q + cq, 16)], bvec,
                                  mask=mq)
            nm = plsc.all_reduce_population_count(mq)[0]
            new.append(jnp.minimum(cq + nm, _QCAP))
        return tuple(new)

    lax.fori_loop(0, _SCAN_VREGS, scan_step,
                  (jnp.int32(0),) * _NQ)

    zero16i = jnp.zeros((16,), jnp.int32)

    def process(buf, q, off, width, cv, ix):
        def rebin_step(k, cnt2):
            base = pl.ds(_QBUF * q + 16 * k, 16)
            av = a_hit[base]
            m2 = (av >= off) & (av < off + width)
            plsc.store_compressed(ca.at[pl.ds(cnt2, 16)], av - off, mask=m2)
            plsc.store_compressed(cb.at[pl.ds(cnt2, 16)], b_hit[base],
                                  mask=m2)
            nm = plsc.all_reduce_population_count(m2)[0]
            return jnp.minimum(cnt2 + nm, _CCAP)

        cnt2 = lax.fori_loop(0, _QB_VREGS, rebin_step, jnp.int32(0))
        for p in range(_CB):
            ca[pl.ds(cnt2 + 16 * p, 16)] = zero16i
            cb[pl.ds(cnt2 + 16 * p, 16)] = dumpb16

        for t in range(_CB):
            jv = ca[pl.ds(16 * t, 16)]
            bv = cb[pl.ds(16 * t, 16)]
            bv32 = bv * _D
            lpos = (16 * t + iota16) * _D
            for c in range(_D):
                cvec = jnp.full((16,), jnp.int32(c), jnp.int32)
                vals = plsc.load_gather(buf, [cvec, jv])
                plsc.store_scatter(cv, [lpos + c], vals)
                plsc.store_scatter(ix, [lpos + c], bv32 + c)

    def scatter(cv, ix):
        pltpu.make_async_copy(cv, outS.at[ix], semS).start()

    def scatter_wait(cv, ix):
        pltpu.make_async_copy(cv, outS.at[ix], semS).wait()

    dump_ix16 = jnp.full((16,), jnp.int32(_B * _D), jnp.int32)
    for k in range(_CVALS // 16):
        ix0[pl.ds(16 * k, 16)] = dump_ix16
        ix1[pl.ds(16 * k, 16)] = dump_ix16
    scatter(cv0, ix0)
    scatter(cv1, ix1)

    def pair_step(g, _):
        for b in range(2):
            i = 2 * g + b
            buf, sem, cv, ix = bufs[b], sems[b], (cv0, cv1)[b], (ix0, ix1)[b]
            pltpu.make_async_copy(
                tabT_hbm.at[:, pl.ds(chunk_off(i), _CW)], buf, sem).wait()
            scatter_wait(cv, ix)
            process(buf, i // 8, chunk_off(i), _CW, cv, ix)
            scatter(cv, ix)
            pltpu.make_async_copy(
                tabT_hbm.at[:, pl.ds(chunk_off(i + 2), _CW)], buf, sem
            ).start()
        return 0

    lax.fori_loop(0, 16, pair_step, 0)
    pltpu.make_async_copy(
        tabT_hbm.at[:, pl.ds(chunk_off(0), _CW)], buf0, sem0).wait()
    pltpu.make_async_copy(
        tabT_hbm.at[:, pl.ds(chunk_off(0), _CW)], buf1, sem1).wait()

    scatter_wait(cv0, ix0)
    scatter_wait(cv1, ix1)
    pltpu.sync_copy(tabT_hbm.at[:, pl.ds(_TAIL, 64)], tailbuf)
    process(tailbuf, _NQ - 1, jnp.int32(_TAIL), 64, cv0, ix0)
    scatter(cv0, ix0)
    scatter_wait(cv0, ix0)
    plsc.subcore_barrier()

    @pl.when(sc == 0)
    def _():
        pltpu.sync_copy(outS.at[pl.ds(32768 * tl, 32768)],
                        p0_hbm.at[pl.ds(32768 * tl, 32768)])

    @pl.when(sc == 1)
    def _():
        pltpu.sync_copy(outS.at[pl.ds(32768 * tl, 32768)],
                        p1_hbm.at[pl.ds(32768 * tl, 32768)])


def kernel(action, table):
    tabT = table.T
    mesh = plsc.VectorSubcoreMesh(core_axis_name="c", subcore_axis_name="s")
    out_sds = jax.ShapeDtypeStruct((_B * _D,), jnp.float32)
    k = pl.kernel(
        _body,
        out_type=(out_sds, out_sds),
        mesh=mesh,
        scratch_types=[
            pltpu.VMEM((_B,), jnp.int32),
            pltpu.VMEM((_D, _CW), jnp.float32),
            pltpu.VMEM((_D, _CW), jnp.float32),
            pltpu.VMEM((_D, 64), jnp.float32),
            pltpu.VMEM((_HBUF,), jnp.int32),
            pltpu.VMEM((_HBUF,), jnp.int32),
            pltpu.VMEM((_CCAP + 48,), jnp.int32),
            pltpu.VMEM((_CCAP + 48,), jnp.int32),
            pltpu.VMEM((_CVALS,), jnp.float32),
            pltpu.VMEM((_CVALS,), jnp.float32),
            pltpu.VMEM((_CVALS,), jnp.int32),
            pltpu.VMEM((_CVALS,), jnp.int32),
            pltpu.VMEM((2048,), jnp.float32),
            pltpu.VMEM_SHARED((_OUTS_WORDS,), jnp.float32),
            pltpu.SemaphoreType.DMA,
            pltpu.SemaphoreType.DMA,
            pltpu.SemaphoreType.DMA,
            pltpu.SemaphoreType.DMA,
        ],
        compiler_params=pltpu.CompilerParams(needs_layout_passes=False),
    )
    p0, p1 = k(action, tabT)
    return (p0 + p1).reshape(_B, _D)
